# Initial kernel scaffold; baseline (speedup 1.0000x reference)
import functools

import jax
import jax.numpy as jnp
from jax import lax
from jax.experimental import pallas as pl
from jax.experimental.pallas import tpu as pltpu

_DELTA_VAR = 0.5
_DELTA_DIST = 1.5
_ALPHA = 1.0
_BETA = 1.0
_GAMMA = 0.001
_K = 16


def _loss_kernel(emb_ref, mask_ref, tot_ref, var_ref, dist_ref, reg_ref,
                 sums, counts, hins, centers, *, B, T, NB):
    K = _K
    phase = pl.program_id(0)
    b = pl.program_id(1)
    n = pl.program_id(2)

    emb_blk = emb_ref[0]
    m = mask_ref[0]
    ids2 = lax.broadcasted_iota(jnp.int32, (K, T), 0)
    onehot = (m == ids2).astype(jnp.float32)

    @pl.when(jnp.logical_and(phase == 0, n == 0))
    def _init_b():
        sums[b] = jnp.zeros_like(sums[b])
        counts[b] = jnp.zeros_like(counts[b])

    @pl.when(phase == 0)
    def _accumulate():
        psum = lax.dot_general(onehot, emb_blk,
                               (((1,), (1,)), ((), ())),
                               preferred_element_type=jnp.float32)
        sums[b] += psum
        counts[b] += jnp.sum(onehot, axis=1, keepdims=True)

    @pl.when(jnp.logical_and(phase == 1,
                             jnp.logical_and(b == 0, n == 0)))
    def _centers():
        for bb in range(B):
            safe = jnp.maximum(counts[bb], 1.0)
            centers[bb] = sums[bb] / safe

    @pl.when(phase == 1)
    def _variance():
        @pl.when(n == 0)
        def _init_h():
            hins[b] = jnp.zeros_like(hins[b])

        c = centers[b]
        pix_c = lax.dot_general(c, onehot,
                                (((0,), (0,)), ((), ())),
                                preferred_element_type=jnp.float32)
        diff = emb_blk - pix_c
        ssum = jnp.sum(diff * diff, axis=0, keepdims=True)
        d = jnp.sqrt(ssum + 1e-12)
        hin = jnp.maximum(d - _DELTA_VAR, 0.0)
        hin = hin * hin
        hins[b] += lax.dot_general(onehot, hin,
                                   (((1,), (1,)), ((), ())),
                                   preferred_element_type=jnp.float32)

    last = jnp.logical_and(phase == 1,
                           jnp.logical_and(b == B - 1, n == NB - 1))

    @pl.when(last)
    def _epilogue():
        ids_k = lax.broadcasted_iota(jnp.int32, (K, 1), 0)
        eyef = (lax.broadcasted_iota(jnp.int32, (K, K), 0)
                == lax.broadcasted_iota(jnp.int32, (K, K), 1)
                ).astype(jnp.float32)
        tri = (lax.broadcasted_iota(jnp.int32, (K, K), 0)
               < lax.broadcasted_iota(jnp.int32, (K, K), 1)
               ).astype(jnp.float32)
        var_t = jnp.float32(0.0)
        dist_t = jnp.float32(0.0)
        reg_t = jnp.float32(0.0)
        valid_b = jnp.float32(0.0)
        for bb in range(B):
            cnt = counts[bb]
            safe = jnp.maximum(cnt, 1.0)
            validf = jnp.logical_and(cnt > 0.0, ids_k > 0
                                     ).astype(jnp.float32)
            nv = jnp.sum(validf)
            per_inst = hins[bb] / safe
            lv = jnp.sum(validf * per_inst) / jnp.maximum(nv, 1.0)

            c = centers[bb]
            G = lax.dot_general(c, c, (((1,), (1,)), ((), ())),
                                preferred_element_type=jnp.float32)
            nrm2_col = jnp.sum(G * eyef, axis=1, keepdims=True)
            nrm2_row = jnp.sum(G * eyef, axis=0, keepdims=True)
            cdist2 = nrm2_col + nrm2_row - 2.0 * G
            cdist = jnp.sqrt(jnp.maximum(cdist2, 0.0) + 1e-12)
            validf_row = jnp.sum(validf * eyef, axis=0,
                                 keepdims=True)
            pairf = validf * validf_row * tri
            hp = jnp.maximum(2.0 * _DELTA_DIST - cdist, 0.0)
            hp = hp * hp
            n_pairs = jnp.sum(pairf)
            ld = jnp.sum(pairf * hp) / jnp.maximum(n_pairs, 1.0)

            nrm = jnp.sqrt(nrm2_col + 1e-12)
            lr = jnp.sum(validf * nrm) / jnp.maximum(nv, 1.0)

            has = (nv > 0.0).astype(jnp.float32)
            var_t = var_t + has * lv
            dist_t = dist_t + has * ld
            reg_t = reg_t + has * lr
            valid_b = valid_b + has
        denom = jnp.maximum(valid_b, 1.0)
        L_var = var_t / denom
        L_dist = dist_t / denom
        L_reg = reg_t / denom
        total = _ALPHA * L_var + _BETA * L_dist + _GAMMA * L_reg
        tot_ref[0, 0] = total
        var_ref[0, 0] = L_var
        dist_ref[0, 0] = L_dist
        reg_ref[0, 0] = L_reg


def kernel(embedding, instance_mask):
    B, E = embedding.shape[0], embedding.shape[1]
    N = embedding.shape[2] * embedding.shape[3]
    K = _K
    T = 16384
    NB = N // T

    emb = embedding.reshape(B, E, N)
    mask3 = instance_mask.reshape(B, 1, N)

    out_shape = [jax.ShapeDtypeStruct((1, 1), jnp.float32)] * 4
    scalar_spec = pl.BlockSpec((1, 1), lambda p, b, n: (0, 0))

    body = functools.partial(_loss_kernel, B=B, T=T, NB=NB)
    outs = pl.pallas_call(
        body,
        grid=(2, B, NB),
        in_specs=[
            pl.BlockSpec((1, E, T), lambda p, b, n: (b, 0, n)),
            pl.BlockSpec((1, 1, T), lambda p, b, n: (b, 0, n)),
        ],
        out_specs=[scalar_spec] * 4,
        out_shape=out_shape,
        scratch_shapes=[
            pltpu.VMEM((B, K, E), jnp.float32),
            pltpu.VMEM((B, K, 1), jnp.float32),
            pltpu.VMEM((B, K, 1), jnp.float32),
            pltpu.VMEM((B, K, E), jnp.float32),
        ],
    )(emb, mask3)
    total, L_var, L_dist, L_reg = [o[0, 0] for o in outs]
    return (total, L_var, L_dist, L_reg)


# speedup vs baseline: 29.2723x; 29.2723x over previous
import functools

import jax
import jax.numpy as jnp
from jax import lax
from jax.experimental import pallas as pl
from jax.experimental.pallas import tpu as pltpu

_DELTA_VAR = 0.5
_DELTA_DIST = 1.5
_ALPHA = 1.0
_BETA = 1.0
_GAMMA = 0.001
_K = 16


def _loss_kernel(emb_ref, mask_ref, tot_ref, var_ref, dist_ref, reg_ref,
                 sums, counts, hins, centers, *, B, T, NB):
    K = _K
    phase = pl.program_id(0)
    b = pl.program_id(1)
    n = pl.program_id(2)

    emb_blk = emb_ref[0]
    m = mask_ref[0]
    ids2 = lax.broadcasted_iota(jnp.int32, (K, T), 0)
    onehot = (m == ids2).astype(jnp.float32)

    @pl.when(jnp.logical_and(phase == 0, n == 0))
    def _init_b():
        sums[b] = jnp.zeros_like(sums[b])
        counts[b] = jnp.zeros_like(counts[b])

    @pl.when(phase == 0)
    def _accumulate():
        psum = lax.dot_general(onehot, emb_blk,
                               (((1,), (1,)), ((), ())),
                               preferred_element_type=jnp.float32)
        sums[b] += psum
        counts[b] += jnp.sum(onehot, axis=1, keepdims=True)

    @pl.when(jnp.logical_and(phase == 1,
                             jnp.logical_and(b == 0, n == 0)))
    def _centers():
        for bb in range(B):
            safe = jnp.maximum(counts[bb], 1.0)
            centers[bb] = sums[bb] / safe

    @pl.when(phase == 1)
    def _variance():
        @pl.when(n == 0)
        def _init_h():
            hins[b] = jnp.zeros_like(hins[b])

        c = centers[b]
        pix_c = lax.dot_general(c, onehot,
                                (((0,), (0,)), ((), ())),
                                preferred_element_type=jnp.float32)
        diff = emb_blk - pix_c
        ssum = jnp.sum(diff * diff, axis=0, keepdims=True)
        d = jnp.sqrt(ssum + 1e-12)
        hin = jnp.maximum(d - _DELTA_VAR, 0.0)
        hin = hin * hin
        hins[b] += lax.dot_general(onehot, hin,
                                   (((1,), (1,)), ((), ())),
                                   preferred_element_type=jnp.float32)

    last = jnp.logical_and(phase == 1,
                           jnp.logical_and(b == B - 1, n == NB - 1))

    @pl.when(last)
    def _epilogue():
        ids_k = lax.broadcasted_iota(jnp.int32, (K, 1), 0)
        eyef = (lax.broadcasted_iota(jnp.int32, (K, K), 0)
                == lax.broadcasted_iota(jnp.int32, (K, K), 1)
                ).astype(jnp.float32)
        tri = (lax.broadcasted_iota(jnp.int32, (K, K), 0)
               < lax.broadcasted_iota(jnp.int32, (K, K), 1)
               ).astype(jnp.float32)
        var_t = jnp.float32(0.0)
        dist_t = jnp.float32(0.0)
        reg_t = jnp.float32(0.0)
        valid_b = jnp.float32(0.0)
        for bb in range(B):
            cnt = counts[bb]
            safe = jnp.maximum(cnt, 1.0)
            validf = jnp.logical_and(cnt > 0.0, ids_k > 0
                                     ).astype(jnp.float32)
            nv = jnp.sum(validf)
            per_inst = hins[bb] / safe
            lv = jnp.sum(validf * per_inst) / jnp.maximum(nv, 1.0)

            c = centers[bb]
            G = lax.dot_general(c, c, (((1,), (1,)), ((), ())),
                                preferred_element_type=jnp.float32)
            nrm2_col = jnp.sum(G * eyef, axis=1, keepdims=True)
            nrm2_row = jnp.sum(G * eyef, axis=0, keepdims=True)
            cdist2 = nrm2_col + nrm2_row - 2.0 * G
            cdist = jnp.sqrt(jnp.maximum(cdist2, 0.0) + 1e-12)
            validf_row = jnp.sum(validf * eyef, axis=0,
                                 keepdims=True)
            pairf = validf * validf_row * tri
            hp = jnp.maximum(2.0 * _DELTA_DIST - cdist, 0.0)
            hp = hp * hp
            n_pairs = jnp.sum(pairf)
            ld = jnp.sum(pairf * hp) / jnp.maximum(n_pairs, 1.0)

            nrm = jnp.sqrt(nrm2_col + 1e-12)
            lr = jnp.sum(validf * nrm) / jnp.maximum(nv, 1.0)

            has = (nv > 0.0).astype(jnp.float32)
            var_t = var_t + has * lv
            dist_t = dist_t + has * ld
            reg_t = reg_t + has * lr
            valid_b = valid_b + has
        denom = jnp.maximum(valid_b, 1.0)
        L_var = var_t / denom
        L_dist = dist_t / denom
        L_reg = reg_t / denom
        total = _ALPHA * L_var + _BETA * L_dist + _GAMMA * L_reg
        tot_ref[...] = jnp.broadcast_to(total, (1, 1))
        var_ref[...] = jnp.broadcast_to(L_var, (1, 1))
        dist_ref[...] = jnp.broadcast_to(L_dist, (1, 1))
        reg_ref[...] = jnp.broadcast_to(L_reg, (1, 1))


def kernel(embedding, instance_mask):
    B, E = embedding.shape[0], embedding.shape[1]
    N = embedding.shape[2] * embedding.shape[3]
    K = _K
    T = 16384
    NB = N // T

    emb = embedding.reshape(B, E, N)
    mask3 = instance_mask.reshape(B, 1, N)

    out_shape = [jax.ShapeDtypeStruct((1, 1), jnp.float32)] * 4
    scalar_spec = pl.BlockSpec((1, 1), lambda p, b, n: (0, 0))

    body = functools.partial(_loss_kernel, B=B, T=T, NB=NB)
    outs = pl.pallas_call(
        body,
        grid=(2, B, NB),
        in_specs=[
            pl.BlockSpec((1, E, T), lambda p, b, n: (b, 0, n)),
            pl.BlockSpec((1, 1, T), lambda p, b, n: (b, 0, n)),
        ],
        out_specs=[scalar_spec] * 4,
        out_shape=out_shape,
        scratch_shapes=[
            pltpu.VMEM((B, K, E), jnp.float32),
            pltpu.VMEM((B, K, 1), jnp.float32),
            pltpu.VMEM((B, K, 1), jnp.float32),
            pltpu.VMEM((B, K, E), jnp.float32),
        ],
    )(emb, mask3)
    total, L_var, L_dist, L_reg = [o[0, 0] for o in outs]
    return (total, L_var, L_dist, L_reg)


# speedup vs baseline: 34.2525x; 1.1701x over previous
import functools

import jax
import jax.numpy as jnp
from jax import lax
from jax.experimental import pallas as pl
from jax.experimental.pallas import tpu as pltpu

_DELTA_VAR = 0.5
_DELTA_DIST = 1.5
_ALPHA = 1.0
_BETA = 1.0
_GAMMA = 0.001
_K = 16


def _loss_kernel(emb_ref, mask_ref, tot_ref, var_ref, dist_ref, reg_ref,
                 sums, counts, hins, centers, *, B, T, NB):
    K = _K
    phase = pl.program_id(0)
    b = pl.program_id(1)
    n = pl.program_id(2)

    emb_blk = emb_ref[0]
    m = mask_ref[0]
    ids2 = lax.broadcasted_iota(jnp.int32, (K, T), 0)
    onehot = (m == ids2).astype(jnp.float32)

    @pl.when(jnp.logical_and(phase == 0, n == 0))
    def _init_b():
        sums[b] = jnp.zeros_like(sums[b])
        counts[b] = jnp.zeros_like(counts[b])

    @pl.when(phase == 0)
    def _accumulate():
        psum = lax.dot_general(onehot, emb_blk,
                               (((1,), (1,)), ((), ())),
                               preferred_element_type=jnp.float32)
        sums[b] += psum
        counts[b] += jnp.sum(onehot, axis=1, keepdims=True)

    @pl.when(jnp.logical_and(phase == 1,
                             jnp.logical_and(b == 0, n == 0)))
    def _centers():
        for bb in range(B):
            safe = jnp.maximum(counts[bb], 1.0)
            centers[bb] = sums[bb] / safe

    @pl.when(phase == 1)
    def _variance():
        @pl.when(n == 0)
        def _init_h():
            hins[b] = jnp.zeros_like(hins[b])

        c = centers[b]
        pix_c = lax.dot_general(c, onehot,
                                (((0,), (0,)), ((), ())),
                                preferred_element_type=jnp.float32)
        diff = emb_blk - pix_c
        ssum = jnp.sum(diff * diff, axis=0, keepdims=True)
        d = jnp.sqrt(ssum + 1e-12)
        hin = jnp.maximum(d - _DELTA_VAR, 0.0)
        hin = hin * hin
        hins[b] += lax.dot_general(onehot, hin,
                                   (((1,), (1,)), ((), ())),
                                   preferred_element_type=jnp.float32)

    last = jnp.logical_and(phase == 1,
                           jnp.logical_and(b == B - 1, n == NB - 1))

    @pl.when(last)
    def _epilogue():
        ids_k = lax.broadcasted_iota(jnp.int32, (K, 1), 0)
        eyef = (lax.broadcasted_iota(jnp.int32, (K, K), 0)
                == lax.broadcasted_iota(jnp.int32, (K, K), 1)
                ).astype(jnp.float32)
        tri = (lax.broadcasted_iota(jnp.int32, (K, K), 0)
               < lax.broadcasted_iota(jnp.int32, (K, K), 1)
               ).astype(jnp.float32)
        var_t = jnp.float32(0.0)
        dist_t = jnp.float32(0.0)
        reg_t = jnp.float32(0.0)
        valid_b = jnp.float32(0.0)
        for bb in range(B):
            cnt = counts[bb]
            safe = jnp.maximum(cnt, 1.0)
            validf = jnp.logical_and(cnt > 0.0, ids_k > 0
                                     ).astype(jnp.float32)
            nv = jnp.sum(validf)
            per_inst = hins[bb] / safe
            lv = jnp.sum(validf * per_inst) / jnp.maximum(nv, 1.0)

            c = centers[bb]
            G = lax.dot_general(c, c, (((1,), (1,)), ((), ())),
                                preferred_element_type=jnp.float32)
            nrm2_col = jnp.sum(G * eyef, axis=1, keepdims=True)
            nrm2_row = jnp.sum(G * eyef, axis=0, keepdims=True)
            cdist2 = nrm2_col + nrm2_row - 2.0 * G
            cdist = jnp.sqrt(jnp.maximum(cdist2, 0.0) + 1e-12)
            validf_row = jnp.sum(validf * eyef, axis=0,
                                 keepdims=True)
            pairf = validf * validf_row * tri
            hp = jnp.maximum(2.0 * _DELTA_DIST - cdist, 0.0)
            hp = hp * hp
            n_pairs = jnp.sum(pairf)
            ld = jnp.sum(pairf * hp) / jnp.maximum(n_pairs, 1.0)

            nrm = jnp.sqrt(nrm2_col + 1e-12)
            lr = jnp.sum(validf * nrm) / jnp.maximum(nv, 1.0)

            has = (nv > 0.0).astype(jnp.float32)
            var_t = var_t + has * lv
            dist_t = dist_t + has * ld
            reg_t = reg_t + has * lr
            valid_b = valid_b + has
        denom = jnp.maximum(valid_b, 1.0)
        L_var = var_t / denom
        L_dist = dist_t / denom
        L_reg = reg_t / denom
        total = _ALPHA * L_var + _BETA * L_dist + _GAMMA * L_reg
        tot_ref[...] = jnp.broadcast_to(total, (1, 1))
        var_ref[...] = jnp.broadcast_to(L_var, (1, 1))
        dist_ref[...] = jnp.broadcast_to(L_dist, (1, 1))
        reg_ref[...] = jnp.broadcast_to(L_reg, (1, 1))


def kernel(embedding, instance_mask):
    B, E = embedding.shape[0], embedding.shape[1]
    N = embedding.shape[2] * embedding.shape[3]
    K = _K
    T = 32768
    NB = N // T

    emb = embedding.reshape(B, E, N)
    mask3 = instance_mask.reshape(B, 1, N)

    out_shape = [jax.ShapeDtypeStruct((1, 1), jnp.float32)] * 4
    scalar_spec = pl.BlockSpec((1, 1), lambda p, b, n: (0, 0))

    body = functools.partial(_loss_kernel, B=B, T=T, NB=NB)
    outs = pl.pallas_call(
        body,
        grid=(2, B, NB),
        in_specs=[
            pl.BlockSpec((1, E, T), lambda p, b, n: (b, 0, n)),
            pl.BlockSpec((1, 1, T), lambda p, b, n: (b, 0, n)),
        ],
        out_specs=[scalar_spec] * 4,
        out_shape=out_shape,
        scratch_shapes=[
            pltpu.VMEM((B, K, E), jnp.float32),
            pltpu.VMEM((B, K, 1), jnp.float32),
            pltpu.VMEM((B, K, 1), jnp.float32),
            pltpu.VMEM((B, K, E), jnp.float32),
        ],
    )(emb, mask3)
    total, L_var, L_dist, L_reg = [o[0, 0] for o in outs]
    return (total, L_var, L_dist, L_reg)


# speedup vs baseline: 36.8056x; 1.0745x over previous
import functools

import jax
import jax.numpy as jnp
from jax import lax
from jax.experimental import pallas as pl
from jax.experimental.pallas import tpu as pltpu

_DELTA_VAR = 0.5
_DELTA_DIST = 1.5
_ALPHA = 1.0
_BETA = 1.0
_GAMMA = 0.001
_K = 16


def _loss_kernel(emb_ref, mask_ref, tot_ref, var_ref, dist_ref, reg_ref,
                 sums, counts, hins, centers, *, B, T, NB):
    K = _K
    phase = pl.program_id(0)
    b = pl.program_id(1)
    n = pl.program_id(2)

    emb_blk = emb_ref[0]
    m = mask_ref[0]
    ids2 = lax.broadcasted_iota(jnp.int32, (K, T), 0)
    onehot = (m == ids2).astype(jnp.float32)

    @pl.when(jnp.logical_and(phase == 0, n == 0))
    def _init_b():
        sums[b] = jnp.zeros_like(sums[b])
        counts[b] = jnp.zeros_like(counts[b])

    @pl.when(phase == 0)
    def _accumulate():
        psum = lax.dot_general(onehot, emb_blk,
                               (((1,), (1,)), ((), ())),
                               preferred_element_type=jnp.float32)
        sums[b] += psum
        counts[b] += jnp.sum(onehot, axis=1, keepdims=True)

    @pl.when(jnp.logical_and(phase == 1,
                             jnp.logical_and(b == 0, n == 0)))
    def _centers():
        for bb in range(B):
            safe = jnp.maximum(counts[bb], 1.0)
            centers[bb] = sums[bb] / safe

    @pl.when(phase == 1)
    def _variance():
        @pl.when(n == 0)
        def _init_h():
            hins[b] = jnp.zeros_like(hins[b])

        c = centers[b]
        pix_c = lax.dot_general(c, onehot,
                                (((0,), (0,)), ((), ())),
                                preferred_element_type=jnp.float32)
        diff = emb_blk - pix_c
        ssum = jnp.sum(diff * diff, axis=0, keepdims=True)
        d = jnp.sqrt(ssum + 1e-12)
        hin = jnp.maximum(d - _DELTA_VAR, 0.0)
        hin = hin * hin
        hins[b] += lax.dot_general(onehot, hin,
                                   (((1,), (1,)), ((), ())),
                                   preferred_element_type=jnp.float32)

    last = jnp.logical_and(phase == 1,
                           jnp.logical_and(b == B - 1, n == NB - 1))

    @pl.when(last)
    def _epilogue():
        ids_k = lax.broadcasted_iota(jnp.int32, (K, 1), 0)
        eyef = (lax.broadcasted_iota(jnp.int32, (K, K), 0)
                == lax.broadcasted_iota(jnp.int32, (K, K), 1)
                ).astype(jnp.float32)
        tri = (lax.broadcasted_iota(jnp.int32, (K, K), 0)
               < lax.broadcasted_iota(jnp.int32, (K, K), 1)
               ).astype(jnp.float32)
        var_t = jnp.float32(0.0)
        dist_t = jnp.float32(0.0)
        reg_t = jnp.float32(0.0)
        valid_b = jnp.float32(0.0)
        for bb in range(B):
            cnt = counts[bb]
            safe = jnp.maximum(cnt, 1.0)
            validf = jnp.logical_and(cnt > 0.0, ids_k > 0
                                     ).astype(jnp.float32)
            nv = jnp.sum(validf)
            per_inst = hins[bb] / safe
            lv = jnp.sum(validf * per_inst) / jnp.maximum(nv, 1.0)

            c = centers[bb]
            G = lax.dot_general(c, c, (((1,), (1,)), ((), ())),
                                preferred_element_type=jnp.float32)
            nrm2_col = jnp.sum(G * eyef, axis=1, keepdims=True)
            nrm2_row = jnp.sum(G * eyef, axis=0, keepdims=True)
            cdist2 = nrm2_col + nrm2_row - 2.0 * G
            cdist = jnp.sqrt(jnp.maximum(cdist2, 0.0) + 1e-12)
            validf_row = jnp.sum(validf * eyef, axis=0,
                                 keepdims=True)
            pairf = validf * validf_row * tri
            hp = jnp.maximum(2.0 * _DELTA_DIST - cdist, 0.0)
            hp = hp * hp
            n_pairs = jnp.sum(pairf)
            ld = jnp.sum(pairf * hp) / jnp.maximum(n_pairs, 1.0)

            nrm = jnp.sqrt(nrm2_col + 1e-12)
            lr = jnp.sum(validf * nrm) / jnp.maximum(nv, 1.0)

            has = (nv > 0.0).astype(jnp.float32)
            var_t = var_t + has * lv
            dist_t = dist_t + has * ld
            reg_t = reg_t + has * lr
            valid_b = valid_b + has
        denom = jnp.maximum(valid_b, 1.0)
        L_var = var_t / denom
        L_dist = dist_t / denom
        L_reg = reg_t / denom
        total = _ALPHA * L_var + _BETA * L_dist + _GAMMA * L_reg
        tot_ref[...] = jnp.broadcast_to(total, (1, 1))
        var_ref[...] = jnp.broadcast_to(L_var, (1, 1))
        dist_ref[...] = jnp.broadcast_to(L_dist, (1, 1))
        reg_ref[...] = jnp.broadcast_to(L_reg, (1, 1))


def kernel(embedding, instance_mask):
    B, E = embedding.shape[0], embedding.shape[1]
    N = embedding.shape[2] * embedding.shape[3]
    K = _K
    T = 65536
    NB = N // T

    emb = embedding.reshape(B, E, N)
    mask3 = instance_mask.reshape(B, 1, N)

    out_shape = [jax.ShapeDtypeStruct((1, 1), jnp.float32)] * 4
    scalar_spec = pl.BlockSpec((1, 1), lambda p, b, n: (0, 0))

    body = functools.partial(_loss_kernel, B=B, T=T, NB=NB)
    outs = pl.pallas_call(
        body,
        grid=(2, B, NB),
        in_specs=[
            pl.BlockSpec((1, E, T), lambda p, b, n: (b, 0, n)),
            pl.BlockSpec((1, 1, T), lambda p, b, n: (b, 0, n)),
        ],
        out_specs=[scalar_spec] * 4,
        out_shape=out_shape,
        scratch_shapes=[
            pltpu.VMEM((B, K, E), jnp.float32),
            pltpu.VMEM((B, K, 1), jnp.float32),
            pltpu.VMEM((B, K, 1), jnp.float32),
            pltpu.VMEM((B, K, E), jnp.float32),
        ],
    )(emb, mask3)
    total, L_var, L_dist, L_reg = [o[0, 0] for o in outs]
    return (total, L_var, L_dist, L_reg)
